# baseline (speedup 1.0000x reference)
import functools

import jax
import jax.numpy as jnp
from jax.experimental import pallas as pl
from jax.experimental.pallas import tpu as pltpu


def _rope(x, c, s):
    half = x.shape[-1] // 2
    rot = jnp.concatenate([-x[..., half:], x[..., :half]], axis=-1)
    return x * c + rot * s


def _fused_kernel(x_ref, tt_ref, cos_ref, sin_ref, Wq_ref, bq_ref, Wk_ref,
                  bk_ref, Wv_ref, bv_ref, Wo_ref, o_ref, ksc, vsc, asc, *,
                  bq_blk, seq, h, kv, dh, scale):
    i = pl.program_id(0)
    nrep = h // kv
    x = x_ref[...]
    sel = tt_ref[...] == 1
    c = cos_ref[...]
    s = sin_ref[...]

    def proj(W_ref, b_ref):
        y0 = jnp.dot(x, W_ref[0], preferred_element_type=jnp.float32)
        y1 = jnp.dot(x, W_ref[1], preferred_element_type=jnp.float32)
        return jnp.where(sel, y1 + b_ref[1:2, :], y0 + b_ref[0:1, :])

    @pl.when(i == 0)
    def _zero_scratch():
        vsc[...] = jnp.zeros_like(vsc)

    q = proj(Wq_ref, bq_ref)
    k = proj(Wk_ref, bk_ref)
    vsc[pl.ds(i * bq_blk, bq_blk), :] = proj(Wv_ref, bv_ref).astype(
        jnp.bfloat16)
    for g in range(kv):
        ksc[pl.ds(i * bq_blk, bq_blk), g * dh:(g + 1) * dh] = (
            _rope(k[:, g * dh:(g + 1) * dh], c, s).astype(jnp.bfloat16))

    row = i * bq_blk + jax.lax.broadcasted_iota(jnp.int32, (bq_blk, seq), 0)
    col = jax.lax.broadcasted_iota(jnp.int32, (bq_blk, seq), 1)
    mask = col <= row

    for hh in range(h):
        g = hh // nrep
        qh = _rope(q[:, hh * dh:(hh + 1) * dh], c, s).astype(jnp.bfloat16)
        kg = ksc[:, g * dh:(g + 1) * dh]
        vg = vsc[:, g * dh:(g + 1) * dh]
        sc = jnp.dot(qh, kg.T, preferred_element_type=jnp.float32) * scale
        sc = jnp.where(mask, sc, -jnp.inf)
        m = jnp.max(sc, axis=-1, keepdims=True)
        p = jnp.exp(sc - m)
        l = jnp.sum(p, axis=-1, keepdims=True)
        pb = (p / l).astype(jnp.bfloat16)
        asc[:, hh * dh:(hh + 1) * dh] = jnp.dot(
            pb, vg, preferred_element_type=jnp.float32).astype(jnp.bfloat16)

    attn = asc[...]
    y0 = jnp.dot(attn, Wo_ref[0], preferred_element_type=jnp.float32)
    y1 = jnp.dot(attn, Wo_ref[1], preferred_element_type=jnp.float32)
    o_ref[...] = jnp.where(sel, y1, y0)


def kernel(hidden_states, token_types, cos, sin, Wq, bq, Wk, bk, Wv, bv, Wo):
    bsz, seq, d = hidden_states.shape
    dh = cos.shape[-1]
    h = Wq.shape[2] // dh
    kv = Wk.shape[2] // dh
    scale = 1.0 / float(dh) ** 0.5

    x = hidden_states.reshape(seq, d).astype(jnp.bfloat16)
    tt = token_types.reshape(seq, 1).astype(jnp.int32)
    cs = cos.reshape(seq, dh)
    sn = sin.reshape(seq, dh)
    Wqb = Wq.astype(jnp.bfloat16)
    Wkb = Wk.astype(jnp.bfloat16)
    Wvb = Wv.astype(jnp.bfloat16)
    Wob = Wo.astype(jnp.bfloat16)

    BQ = 256
    nq = seq // BQ
    full3 = lambda shp: pl.BlockSpec(shp, lambda i: (0, 0, 0))
    full2 = lambda shp: pl.BlockSpec(shp, lambda i: (0, 0))

    out = pl.pallas_call(
        functools.partial(_fused_kernel, bq_blk=BQ, seq=seq, h=h, kv=kv,
                          dh=dh, scale=scale),
        grid=(nq,),
        in_specs=[
            pl.BlockSpec((BQ, d), lambda i: (i, 0)),
            pl.BlockSpec((BQ, 1), lambda i: (i, 0)),
            pl.BlockSpec((BQ, dh), lambda i: (i, 0)),
            pl.BlockSpec((BQ, dh), lambda i: (i, 0)),
            full3(Wqb.shape), full2(bq.shape),
            full3(Wkb.shape), full2(bk.shape),
            full3(Wvb.shape), full2(bv.shape),
            full3(Wob.shape),
        ],
        out_specs=pl.BlockSpec((BQ, d), lambda i: (i, 0)),
        out_shape=jax.ShapeDtypeStruct((seq, d), jnp.float32),
        scratch_shapes=[
            pltpu.VMEM((seq, kv * dh), jnp.bfloat16),
            pltpu.VMEM((seq, kv * dh), jnp.bfloat16),
            pltpu.VMEM((BQ, h * dh), jnp.bfloat16),
        ],
        compiler_params=pltpu.CompilerParams(
            vmem_limit_bytes=63 * 1024 * 1024),
    )(x, tt, cs, sn, Wqb, bq, Wkb, bk, Wvb, bv, Wob)

    return out.reshape(bsz, seq, d)
